# baseline (speedup 1.0000x reference)
import functools

import jax
import jax.numpy as jnp
from jax import lax
from jax.experimental import pallas as pl
from jax.experimental.pallas import tpu as pltpu
from jax.experimental.pallas import tpu_sc as plsc

N = 10000
E = 320000
D = 128
K = 4
C_AGG = 0.5
C_H0 = 0.5

NC = 2
NS = 16
DH = D // NC

ROWS_PER_TILE = 640
NPAD = NS * ROWS_PER_TILE
SENT = NPAD - 1
NQ = 128
NQCHUNKS = ROWS_PER_TILE // NQ

EC = 128
ECHUNKS = 160
EPT = EC * ECHUNKS
E_PAD = EPT * NS

UNROLL = 16
GROUPS = ECHUNKS // UNROLL
PDIST = 8

MM_BLOCK = 256


def _rsqrt_pos(d):
  i = plsc.bitcast(d, jnp.int32)
  i = 0x5F3759DF - (i >> 1)
  r = plsc.bitcast(i, jnp.float32)
  for _ in range(4):
    r = r * (1.5 - 0.5 * d * r * r)
  return jnp.where(d > 0.0, r, 1.0)


def _mm_body(x_ref, w_ref, b_ref, o_ref):
  o_ref[0] = (
      jnp.dot(x_ref[...], w_ref[0], preferred_element_type=jnp.float32)
      + b_ref[0]
  )


def _sc_body(h0_hbm, e_hbm, out_hbm,
             hs_sp, agg_sp, degs_sp, degd_sp,
             ibuf, gbuf0, gbuf1, gbuf2, gbuf3, nbuf,
             zbuf, zvec, ones_v, ns_v, nd_v,
             isem, gsem, ssem):
  cid = lax.axis_index("c")
  sid = lax.axis_index("s")
  n0 = sid * ROWS_PER_TILE
  gbufs = (gbuf0, gbuf1, gbuf2, gbuf3)

  zeros16 = jnp.zeros((16,), jnp.float32)
  ones16 = jnp.ones((16,), jnp.float32)

  for r in range(8):
    for k in range(DH // 16):
      zbuf[r, pl.ds(16 * k, 16)] = zeros16

  def _fill_zvec(q, c):
    zvec[pl.ds(16 * q, 16)] = zeros16
    return c
  lax.fori_loop(0, ROWS_PER_TILE // 16, _fill_zvec, 0)

  for k in range(EC // 16):
    ones_v[pl.ds(16 * k, 16)] = ones16

  rows640 = pl.ds(n0, ROWS_PER_TILE)

  def _zero_agg(q, c):
    pltpu.async_copy(zbuf, agg_sp.at[pl.ds(n0 + 8 * q, 8)], gsem.at[0])
    return c
  lax.fori_loop(0, ROWS_PER_TILE // 8, _zero_agg, 0)
  pltpu.async_copy(zvec, degs_sp.at[rows640], gsem.at[1])
  pltpu.async_copy(zvec, degd_sp.at[rows640], gsem.at[1])

  def _zero_agg_wait(q, c):
    pltpu.make_async_copy(zbuf, agg_sp.at[pl.ds(n0, 8)], gsem.at[0]).wait()
    return c
  lax.fori_loop(0, ROWS_PER_TILE // 8, _zero_agg_wait, 0)
  pltpu.make_async_copy(zvec, degs_sp.at[rows640], gsem.at[1]).wait()
  pltpu.make_async_copy(zvec, degd_sp.at[rows640], gsem.at[1]).wait()

  plsc.subcore_barrier()

  def _idx_start(jj, u):
    pltpu.async_copy(e_hbm.at[0, sid, jj], ibuf.at[u, 0], isem.at[u])
    pltpu.async_copy(e_hbm.at[1, sid, jj], ibuf.at[u, 1], isem.at[u])

  def _idx_wait(jj, u):
    pltpu.make_async_copy(
        e_hbm.at[0, sid, jj], ibuf.at[u, 0], isem.at[u]).wait()
    pltpu.make_async_copy(
        e_hbm.at[1, sid, jj], ibuf.at[u, 1], isem.at[u]).wait()

  for s in range(PDIST):
    _idx_start(s, s)

  def _deg_body(g, c):
    for u in range(UNROLL):
      j = UNROLL * g + u
      b = u % 4
      if u < 4:
        @pl.when(g > 0)
        def _():
          pltpu.make_async_copy(
              ones_v, degs_sp.at[ibuf.at[(u - 4) % UNROLL, 0]],
              gsem.at[b]).wait()
          pltpu.make_async_copy(
              ones_v, degd_sp.at[ibuf.at[(u - 4) % UNROLL, 1]],
              ssem.at[b]).wait()
      else:
        pltpu.make_async_copy(
            ones_v, degs_sp.at[ibuf.at[u - 4, 0]], gsem.at[b]).wait()
        pltpu.make_async_copy(
            ones_v, degd_sp.at[ibuf.at[u - 4, 1]], ssem.at[b]).wait()
      if u < UNROLL - PDIST:
        _idx_start(j + PDIST, u + PDIST)
      else:
        @pl.when(g < GROUPS - 1)
        def _():
          _idx_start(j + PDIST, u + PDIST - UNROLL)
      _idx_wait(j, u)
      pltpu.async_copy(ones_v, degs_sp.at[ibuf.at[u, 0]], gsem.at[b],
                       add=True)
      pltpu.async_copy(ones_v, degd_sp.at[ibuf.at[u, 1]], ssem.at[b],
                       add=True)
    return c
  lax.fori_loop(0, GROUPS, _deg_body, 0)
  for b in range(4):
    u = UNROLL - 4 + b
    pltpu.make_async_copy(
        ones_v, degs_sp.at[ibuf.at[u, 0]], gsem.at[b]).wait()
    pltpu.make_async_copy(
        ones_v, degd_sp.at[ibuf.at[u, 1]], ssem.at[b]).wait()

  plsc.subcore_barrier()

  pltpu.sync_copy(degs_sp.at[rows640], ns_v.at[pl.ds(0, ROWS_PER_TILE)])
  pltpu.sync_copy(degd_sp.at[rows640], nd_v.at[pl.ds(0, ROWS_PER_TILE)])

  def _norm_body(q, c):
    sl = pl.ds(16 * q, 16)
    ns_v[sl] = _rsqrt_pos(ns_v[sl])
    nd_v[sl] = _rsqrt_pos(nd_v[sl])
    return c
  lax.fori_loop(0, ROWS_PER_TILE // 16, _norm_body, 0)

  h0bufs = (gbuf0, gbuf1)

  def _h0_start(q, p):
    pltpu.async_copy(h0_hbm.at[cid, pl.ds(n0 + NQ * q, NQ)], h0bufs[p],
                     isem.at[2 + p])

  def _h0_wait(q, p):
    pltpu.make_async_copy(h0_hbm.at[cid, pl.ds(n0, NQ)], h0bufs[p],
                          isem.at[2 + p]).wait()

  def _hs_wait():
    pltpu.make_async_copy(nbuf, hs_sp.at[pl.ds(n0, NQ)], isem.at[4]).wait()

  _h0_start(0, 0)
  for q in range(NQCHUNKS):
    p = q % 2
    if q + 1 < NQCHUNKS:
      _h0_start(q + 1, 1 - p)
    _h0_wait(q, p)

    def _hs0_body(r, c, q=q, p=p):
      ns_s = ns_v[pl.ds(NQ * q + r, 16)][0]
      for k in range(DH // 16):
        sl = pl.ds(16 * k, 16)
        nbuf[r, sl] = h0bufs[p][r, sl] * ns_s
      return c
    lax.fori_loop(0, NQ, _hs0_body, 0)
    pltpu.sync_copy(nbuf, hs_sp.at[pl.ds(n0 + NQ * q, NQ)])

  plsc.subcore_barrier()

  def _gather_start(jj, u, b):
    pltpu.async_copy(hs_sp.at[ibuf.at[u, 0]], gbufs[b], gsem.at[b])

  def _gather_wait(jj, u, b):
    pltpu.make_async_copy(
        hs_sp.at[ibuf.at[u, 0]], gbufs[b], gsem.at[b]).wait()

  def _scatter_start(jj, u, b):
    pltpu.async_copy(gbufs[b], agg_sp.at[ibuf.at[u, 1]], ssem.at[b],
                     add=True)

  def _scatter_wait(jj, u, b):
    pltpu.make_async_copy(
        gbufs[b], agg_sp.at[ibuf.at[u, 1]], ssem.at[b]).wait()

  for t in range(K):
    for s in range(PDIST):
      _idx_start(s, s)

    def _edge_body(g, c):
      for u in range(UNROLL):
        j = UNROLL * g + u
        b = u % 4
        if u < 4:
          @pl.when(g > 0)
          def _():
            _scatter_wait(j - 4, (u - 4) % UNROLL, b)
        else:
          _scatter_wait(j - 4, u - 4, b)
        if u < UNROLL - PDIST:
          _idx_start(j + PDIST, u + PDIST)
        else:
          @pl.when(g < GROUPS - 1)
          def _():
            _idx_start(j + PDIST, u + PDIST - UNROLL)
        _idx_wait(j, u)
        _gather_start(j, u, b)
        _gather_wait(j, u, b)
        _scatter_start(j, u, b)
      return c

    lax.fori_loop(0, GROUPS, _edge_body, 0)
    for b in range(4):
      _scatter_wait(ECHUNKS - 4 + b, UNROLL - 4 + b, b)

    plsc.subcore_barrier()

    nbufs = (nbuf, gbuf3)

    if t < K - 1:
      def _rezero(r, c):
        for k in range(DH // 16):
          gbuf2[r, pl.ds(16 * k, 16)] = zeros16
        return c
      lax.fori_loop(0, NQ, _rezero, 0)

    def _agg_start(q, p):
      pltpu.async_copy(agg_sp.at[pl.ds(n0 + NQ * q, NQ)], nbufs[p],
                       isem.at[p])

    def _agg_wait(q, p):
      pltpu.make_async_copy(agg_sp.at[pl.ds(n0, NQ)], nbufs[p],
                            isem.at[p]).wait()

    _agg_start(0, 0)
    _h0_start(0, 0)
    for q in range(NQCHUNKS):
      p = q % 2
      rows = pl.ds(n0 + NQ * q, NQ)
      if q >= 2:
        pass
      if q + 1 < NQCHUNKS:
        if q >= 1:
          if t < K - 1:
            pltpu.make_async_copy(
                nbufs[1 - p], hs_sp.at[pl.ds(n0, NQ)],
                isem.at[4 + (1 - p)]).wait()
        _agg_start(q + 1, 1 - p)
        _h0_start(q + 1, 1 - p)
      _agg_wait(q, p)
      _h0_wait(q, p)

      def _node_body(r, c, q=q, t=t, p=p):
        nd_s = nd_v[pl.ds(NQ * q + r, 16)][0]
        ns_s = ns_v[pl.ds(NQ * q + r, 16)][0]
        for k in range(DH // 16):
          sl = pl.ds(16 * k, 16)
          hn = C_AGG * nd_s * nbufs[p][r, sl] + C_H0 * h0bufs[p][r, sl]
          if t < K - 1:
            nbufs[p][r, sl] = hn * ns_s
          else:
            nbufs[p][r, sl] = hn
        return c
      lax.fori_loop(0, NQ, _node_body, 0)

      if t < K - 1:
        pltpu.async_copy(gbuf2, agg_sp.at[rows], isem.at[6])
        pltpu.async_copy(nbufs[p], hs_sp.at[rows], isem.at[4 + p])
      else:
        cols = pl.ds(cid * DH, DH)
        nrows15 = min(max(N - (NS - 1) * ROWS_PER_TILE - NQ * q, 0), NQ)
        @pl.when(sid < NS - 1)
        def _(q=q, cols=cols, p=p, rows=rows):
          pltpu.sync_copy(nbufs[p], out_hbm.at[rows, cols])
        if nrows15 > 0:
          @pl.when(sid == NS - 1)
          def _(q=q, cols=cols, nrows15=nrows15, p=p):
            pltpu.sync_copy(
                nbufs[p].at[pl.ds(0, nrows15)],
                out_hbm.at[pl.ds((NS - 1) * ROWS_PER_TILE + NQ * q,
                                 nrows15), cols])

    if t < K - 1:
      for p in (NQCHUNKS % 2, 1 - NQCHUNKS % 2):
        pltpu.make_async_copy(
            nbufs[p], hs_sp.at[pl.ds(n0, NQ)], isem.at[4 + p]).wait()
      for _z in range(NQCHUNKS):
        pltpu.make_async_copy(
            gbuf2, agg_sp.at[pl.ds(n0, NQ)], isem.at[6]).wait()
      plsc.subcore_barrier()


@jax.jit
def kernel(x, edge_index, W, b):
  x_pad = jnp.zeros((NPAD, D), jnp.float32).at[:N].set(x)
  w_split = W.reshape(D, NC, DH).transpose(1, 0, 2)
  b_split = b.reshape(1, NC, DH).transpose(1, 0, 2)
  h0_split = pl.pallas_call(
      _mm_body,
      grid=(NPAD // MM_BLOCK, NC),
      in_specs=[
          pl.BlockSpec((MM_BLOCK, D), lambda i, c: (i, 0)),
          pl.BlockSpec((1, D, DH), lambda i, c: (c, 0, 0)),
          pl.BlockSpec((1, 1, DH), lambda i, c: (c, 0, 0)),
      ],
      out_specs=pl.BlockSpec((1, MM_BLOCK, DH), lambda i, c: (c, i, 0)),
      out_shape=jax.ShapeDtypeStruct((NC, NPAD, DH), jnp.float32),
  )(x_pad, w_split, b_split)

  e4 = jnp.pad(edge_index, ((0, 0), (0, E_PAD - E)),
               constant_values=SENT).reshape(2, NS, ECHUNKS, EC)

  mesh = plsc.VectorSubcoreMesh(
      core_axis_name="c", subcore_axis_name="s",
      num_cores=NC, num_subcores=NS)

  sc = pl.kernel(
      _sc_body,
      out_type=jax.ShapeDtypeStruct((N, D), jnp.float32),
      mesh=mesh,
      compiler_params=pltpu.CompilerParams(
          needs_layout_passes=False, use_tc_tiling_on_sc=False),
      scratch_types=[
          pltpu.VMEM_SHARED((NPAD, DH), jnp.float32),
          pltpu.VMEM_SHARED((NPAD, DH), jnp.float32),
          pltpu.VMEM_SHARED((NPAD,), jnp.float32),
          pltpu.VMEM_SHARED((NPAD,), jnp.float32),
          pltpu.VMEM((UNROLL, 2, EC), jnp.int32),
          pltpu.VMEM((EC, DH), jnp.float32),
          pltpu.VMEM((EC, DH), jnp.float32),
          pltpu.VMEM((EC, DH), jnp.float32),
          pltpu.VMEM((EC, DH), jnp.float32),
          pltpu.VMEM((NQ, DH), jnp.float32),
          pltpu.VMEM((8, DH), jnp.float32),
          pltpu.VMEM((ROWS_PER_TILE,), jnp.float32),
          pltpu.VMEM((EC,), jnp.float32),
          pltpu.VMEM((ROWS_PER_TILE + 16,), jnp.float32),
          pltpu.VMEM((ROWS_PER_TILE + 16,), jnp.float32),
          pltpu.SemaphoreType.DMA((UNROLL,)),
          pltpu.SemaphoreType.DMA((4,)),
          pltpu.SemaphoreType.DMA((4,)),
      ],
  )

  return sc(h0_split, e4)


# speedup vs baseline: 1.0722x; 1.0722x over previous
import functools

import jax
import jax.numpy as jnp
from jax import lax
from jax.experimental import pallas as pl
from jax.experimental.pallas import tpu as pltpu
from jax.experimental.pallas import tpu_sc as plsc

N = 10000
E = 320000
D = 128
K = 4
C_AGG = 0.5
C_H0 = 0.5

NC = 2
NS = 16
DH = D // NC

ROWS_PER_TILE = 640
NPAD = NS * ROWS_PER_TILE
SENT = NPAD - 1
NQ = 128
NQCHUNKS = ROWS_PER_TILE // NQ

EC = 128
ECHUNKS = 160
EPT = EC * ECHUNKS
E_PAD = EPT * NS

UNROLL = 16
GROUPS = ECHUNKS // UNROLL
PDIST = 8

MM_BLOCK = 256


def _rsqrt_pos(d):
  i = plsc.bitcast(d, jnp.int32)
  i = 0x5F3759DF - (i >> 1)
  r = plsc.bitcast(i, jnp.float32)
  for _ in range(4):
    r = r * (1.5 - 0.5 * d * r * r)
  return jnp.where(d > 0.0, r, 1.0)


def _mm_body(x_ref, w_ref, b_ref, o_ref):
  o_ref[0] = (
      jnp.dot(x_ref[...], w_ref[0], preferred_element_type=jnp.float32)
      + b_ref[0]
  )


def _sc_body(h0_hbm, e_hbm, out_hbm,
             hs_sp, agg_sp, degs_sp, degd_sp,
             ibuf, gbuf0, gbuf1, gbuf2, gbuf3, nbuf,
             ones_v, ns_v, nd_v,
             isem, gsem, ssem):
  cid = lax.axis_index("c")
  sid = lax.axis_index("s")
  n0 = sid * ROWS_PER_TILE
  gbufs = (gbuf0, gbuf1, gbuf2, gbuf3)

  zeros16 = jnp.zeros((16,), jnp.float32)
  ones16 = jnp.ones((16,), jnp.float32)

  def _fill_gbuf2(r, c):
    for k in range(DH // 16):
      gbuf2[r, pl.ds(16 * k, 16)] = zeros16
    return c
  lax.fori_loop(0, NQ, _fill_gbuf2, 0)

  def _fill_norms0(q, c):
    ns_v[pl.ds(16 * q, 16)] = zeros16
    nd_v[pl.ds(16 * q, 16)] = zeros16
    return c
  lax.fori_loop(0, ROWS_PER_TILE // 16, _fill_norms0, 0)

  for k in range(EC // 16):
    ones_v[pl.ds(16 * k, 16)] = ones16

  rows640 = pl.ds(n0, ROWS_PER_TILE)
  nsl640 = pl.ds(0, ROWS_PER_TILE)

  for q5 in range(NQCHUNKS):
    pltpu.async_copy(gbuf2, agg_sp.at[pl.ds(n0 + NQ * q5, NQ)], gsem.at[0])
  pltpu.async_copy(ns_v.at[nsl640], degs_sp.at[rows640], gsem.at[1])
  pltpu.async_copy(nd_v.at[nsl640], degd_sp.at[rows640], gsem.at[1])

  for q5 in range(NQCHUNKS):
    pltpu.make_async_copy(gbuf2, agg_sp.at[pl.ds(n0, NQ)],
                          gsem.at[0]).wait()
  pltpu.make_async_copy(ns_v.at[nsl640], degs_sp.at[rows640],
                        gsem.at[1]).wait()
  pltpu.make_async_copy(nd_v.at[nsl640], degd_sp.at[rows640],
                        gsem.at[1]).wait()

  plsc.subcore_barrier()

  def _idx_start(jj, u):
    pltpu.async_copy(e_hbm.at[0, sid, jj], ibuf.at[u, 0], isem.at[u])
    pltpu.async_copy(e_hbm.at[1, sid, jj], ibuf.at[u, 1], isem.at[u])

  def _idx_wait(jj, u):
    pltpu.make_async_copy(
        e_hbm.at[0, sid, jj], ibuf.at[u, 0], isem.at[u]).wait()
    pltpu.make_async_copy(
        e_hbm.at[1, sid, jj], ibuf.at[u, 1], isem.at[u]).wait()

  for s in range(PDIST):
    _idx_start(s, s)

  def _deg_body(g, c):
    for u in range(UNROLL):
      j = UNROLL * g + u
      b = u % 4
      if u < 4:
        @pl.when(g > 0)
        def _():
          pltpu.make_async_copy(
              ones_v, degs_sp.at[ibuf.at[(u - 4) % UNROLL, 0]],
              gsem.at[b]).wait()
          pltpu.make_async_copy(
              ones_v, degd_sp.at[ibuf.at[(u - 4) % UNROLL, 1]],
              ssem.at[b]).wait()
      else:
        pltpu.make_async_copy(
            ones_v, degs_sp.at[ibuf.at[u - 4, 0]], gsem.at[b]).wait()
        pltpu.make_async_copy(
            ones_v, degd_sp.at[ibuf.at[u - 4, 1]], ssem.at[b]).wait()
      if u < UNROLL - PDIST:
        _idx_start(j + PDIST, u + PDIST)
      else:
        @pl.when(g < GROUPS - 1)
        def _():
          _idx_start(j + PDIST, u + PDIST - UNROLL)
      _idx_wait(j, u)
      pltpu.async_copy(ones_v, degs_sp.at[ibuf.at[u, 0]], gsem.at[b],
                       add=True)
      pltpu.async_copy(ones_v, degd_sp.at[ibuf.at[u, 1]], ssem.at[b],
                       add=True)
    return c
  lax.fori_loop(0, GROUPS, _deg_body, 0)
  for b in range(4):
    u = UNROLL - 4 + b
    pltpu.make_async_copy(
        ones_v, degs_sp.at[ibuf.at[u, 0]], gsem.at[b]).wait()
    pltpu.make_async_copy(
        ones_v, degd_sp.at[ibuf.at[u, 1]], ssem.at[b]).wait()

  plsc.subcore_barrier()

  pltpu.sync_copy(degs_sp.at[rows640], ns_v.at[pl.ds(0, ROWS_PER_TILE)])
  pltpu.sync_copy(degd_sp.at[rows640], nd_v.at[pl.ds(0, ROWS_PER_TILE)])

  def _norm_body(q, c):
    sl = pl.ds(16 * q, 16)
    ns_v[sl] = _rsqrt_pos(ns_v[sl])
    nd_v[sl] = _rsqrt_pos(nd_v[sl])
    return c
  lax.fori_loop(0, ROWS_PER_TILE // 16, _norm_body, 0)

  h0bufs = (gbuf0, gbuf1)

  def _h0_start(q, p):
    pltpu.async_copy(h0_hbm.at[cid, pl.ds(n0 + NQ * q, NQ)], h0bufs[p],
                     isem.at[2 + p])

  def _h0_wait(q, p):
    pltpu.make_async_copy(h0_hbm.at[cid, pl.ds(n0, NQ)], h0bufs[p],
                          isem.at[2 + p]).wait()

  def _hs_wait():
    pltpu.make_async_copy(nbuf, hs_sp.at[pl.ds(n0, NQ)], isem.at[4]).wait()

  _h0_start(0, 0)
  for q in range(NQCHUNKS):
    p = q % 2
    if q + 1 < NQCHUNKS:
      _h0_start(q + 1, 1 - p)
    _h0_wait(q, p)

    def _hs0_body(g8, c, q=q, p=p):
      nsv = ns_v[pl.ds(NQ * q + 8 * g8, 16)]
      for i in range(8):
        r = 8 * g8 + i
        ns_s = nsv[i]
        for k in range(DH // 16):
          sl = pl.ds(16 * k, 16)
          nbuf[r, sl] = h0bufs[p][r, sl] * ns_s
      return c
    lax.fori_loop(0, NQ // 8, _hs0_body, 0)
    pltpu.sync_copy(nbuf, hs_sp.at[pl.ds(n0 + NQ * q, NQ)])

  plsc.subcore_barrier()

  def _gather_start(jj, u, b):
    pltpu.async_copy(hs_sp.at[ibuf.at[u, 0]], gbufs[b], gsem.at[b])

  def _gather_wait(jj, u, b):
    pltpu.make_async_copy(
        hs_sp.at[ibuf.at[u, 0]], gbufs[b], gsem.at[b]).wait()

  def _scatter_start(jj, u, b):
    pltpu.async_copy(gbufs[b], agg_sp.at[ibuf.at[u, 1]], ssem.at[b],
                     add=True)

  def _scatter_wait(jj, u, b):
    pltpu.make_async_copy(
        gbufs[b], agg_sp.at[ibuf.at[u, 1]], ssem.at[b]).wait()

  def _one_round(last):
    t = K - 1 if last else 0
    for s in range(PDIST):
      _idx_start(s, s)

    def _edge_body(g, c):
      for u in range(UNROLL):
        j = UNROLL * g + u
        b = u % 4
        if u < 4:
          @pl.when(g > 0)
          def _():
            _scatter_wait(j - 4, (u - 4) % UNROLL, b)
        else:
          _scatter_wait(j - 4, u - 4, b)
        if u < UNROLL - PDIST:
          _idx_start(j + PDIST, u + PDIST)
        else:
          @pl.when(g < GROUPS - 1)
          def _():
            _idx_start(j + PDIST, u + PDIST - UNROLL)
        _idx_wait(j, u)
        _gather_start(j, u, b)
        _gather_wait(j, u, b)
        _scatter_start(j, u, b)
      return c

    lax.fori_loop(0, GROUPS, _edge_body, 0)
    for b in range(4):
      _scatter_wait(ECHUNKS - 4 + b, UNROLL - 4 + b, b)

    plsc.subcore_barrier()

    nbufs = (nbuf, gbuf3)

    if t < K - 1:
      def _rezero(r, c):
        for k in range(DH // 16):
          gbuf2[r, pl.ds(16 * k, 16)] = zeros16
        return c
      lax.fori_loop(0, NQ, _rezero, 0)

    def _agg_start(q, p):
      pltpu.async_copy(agg_sp.at[pl.ds(n0 + NQ * q, NQ)], nbufs[p],
                       isem.at[p])

    def _agg_wait(q, p):
      pltpu.make_async_copy(agg_sp.at[pl.ds(n0, NQ)], nbufs[p],
                            isem.at[p]).wait()

    _agg_start(0, 0)
    _h0_start(0, 0)
    for q in range(NQCHUNKS):
      p = q % 2
      rows = pl.ds(n0 + NQ * q, NQ)
      if q >= 2:
        pass
      if q + 1 < NQCHUNKS:
        if q >= 1:
          if t < K - 1:
            pltpu.make_async_copy(
                nbufs[1 - p], hs_sp.at[pl.ds(n0, NQ)],
                isem.at[4 + (1 - p)]).wait()
        _agg_start(q + 1, 1 - p)
        _h0_start(q + 1, 1 - p)
      _agg_wait(q, p)
      _h0_wait(q, p)

      def _node_body(g8, c, q=q, t=t, p=p):
        ndv = nd_v[pl.ds(NQ * q + 8 * g8, 16)]
        nsv = ns_v[pl.ds(NQ * q + 8 * g8, 16)]
        for i in range(8):
          r = 8 * g8 + i
          nd_s = C_AGG * ndv[i]
          ns_s = nsv[i]
          for k in range(DH // 16):
            sl = pl.ds(16 * k, 16)
            hn = nd_s * nbufs[p][r, sl] + C_H0 * h0bufs[p][r, sl]
            if t < K - 1:
              nbufs[p][r, sl] = hn * ns_s
            else:
              nbufs[p][r, sl] = hn
        return c
      lax.fori_loop(0, NQ // 8, _node_body, 0)

      if t < K - 1:
        pltpu.async_copy(gbuf2, agg_sp.at[rows], isem.at[6])
        pltpu.async_copy(nbufs[p], hs_sp.at[rows], isem.at[4 + p])
      else:
        cols = pl.ds(cid * DH, DH)
        nrows15 = min(max(N - (NS - 1) * ROWS_PER_TILE - NQ * q, 0), NQ)
        @pl.when(sid < NS - 1)
        def _(q=q, cols=cols, p=p, rows=rows):
          pltpu.sync_copy(nbufs[p], out_hbm.at[rows, cols])
        if nrows15 > 0:
          @pl.when(sid == NS - 1)
          def _(q=q, cols=cols, nrows15=nrows15, p=p):
            pltpu.sync_copy(
                nbufs[p].at[pl.ds(0, nrows15)],
                out_hbm.at[pl.ds((NS - 1) * ROWS_PER_TILE + NQ * q,
                                 nrows15), cols])

    if t < K - 1:
      for p in (NQCHUNKS % 2, 1 - NQCHUNKS % 2):
        pltpu.make_async_copy(
            nbufs[p], hs_sp.at[pl.ds(n0, NQ)], isem.at[4 + p]).wait()
      for _z in range(NQCHUNKS):
        pltpu.make_async_copy(
            gbuf2, agg_sp.at[pl.ds(n0, NQ)], isem.at[6]).wait()
      plsc.subcore_barrier()

  def _round_body(t, c):
    _one_round(False)
    return c
  lax.fori_loop(0, K - 1, _round_body, 0)
  _one_round(True)


@jax.jit
def kernel(x, edge_index, W, b):
  w_split = W.reshape(D, NC, DH).transpose(1, 0, 2)
  b_split = b.reshape(1, NC, DH).transpose(1, 0, 2)
  h0_split = pl.pallas_call(
      _mm_body,
      grid=(NPAD // MM_BLOCK, NC),
      in_specs=[
          pl.BlockSpec((MM_BLOCK, D), lambda i, c: (i, 0)),
          pl.BlockSpec((1, D, DH), lambda i, c: (c, 0, 0)),
          pl.BlockSpec((1, 1, DH), lambda i, c: (c, 0, 0)),
      ],
      out_specs=pl.BlockSpec((1, MM_BLOCK, DH), lambda i, c: (c, i, 0)),
      out_shape=jax.ShapeDtypeStruct((NC, NPAD, DH), jnp.float32),
  )(x, w_split, b_split)

  e4 = jnp.pad(edge_index, ((0, 0), (0, E_PAD - E)),
               constant_values=SENT).reshape(2, NS, ECHUNKS, EC)

  mesh = plsc.VectorSubcoreMesh(
      core_axis_name="c", subcore_axis_name="s",
      num_cores=NC, num_subcores=NS)

  sc = pl.kernel(
      _sc_body,
      out_type=jax.ShapeDtypeStruct((N, D), jnp.float32),
      mesh=mesh,
      compiler_params=pltpu.CompilerParams(
          needs_layout_passes=False, use_tc_tiling_on_sc=False),
      scratch_types=[
          pltpu.VMEM_SHARED((NPAD, DH), jnp.float32),
          pltpu.VMEM_SHARED((NPAD, DH), jnp.float32),
          pltpu.VMEM_SHARED((NPAD,), jnp.float32),
          pltpu.VMEM_SHARED((NPAD,), jnp.float32),
          pltpu.VMEM((UNROLL, 2, EC), jnp.int32),
          pltpu.VMEM((EC, DH), jnp.float32),
          pltpu.VMEM((EC, DH), jnp.float32),
          pltpu.VMEM((EC, DH), jnp.float32),
          pltpu.VMEM((EC, DH), jnp.float32),
          pltpu.VMEM((NQ, DH), jnp.float32),
          pltpu.VMEM((EC,), jnp.float32),
          pltpu.VMEM((ROWS_PER_TILE + 16,), jnp.float32),
          pltpu.VMEM((ROWS_PER_TILE + 16,), jnp.float32),
          pltpu.SemaphoreType.DMA((UNROLL,)),
          pltpu.SemaphoreType.DMA((4,)),
          pltpu.SemaphoreType.DMA((4,)),
      ],
  )

  return sc(h0_split, e4)


# speedup vs baseline: 1.1565x; 1.0787x over previous
import functools

import jax
import jax.numpy as jnp
from jax import lax
from jax.experimental import pallas as pl
from jax.experimental.pallas import tpu as pltpu
from jax.experimental.pallas import tpu_sc as plsc

N = 10000
E = 320000
D = 128
K = 4
C_AGG = 0.5
C_H0 = 0.5

NC = 2
NS = 16
DH = D // NC

ROWS_PER_TILE = 640
NPAD = NS * ROWS_PER_TILE
SENT = NPAD - 1
NQ = 128
NQCHUNKS = ROWS_PER_TILE // NQ

EC = 128
ECHUNKS = 160
EPT = EC * ECHUNKS
E_PAD = EPT * NS

UNROLL = 16
GROUPS = ECHUNKS // UNROLL
PDIST = 8

MM_BLOCK = 256


def _rsqrt_pos(d):
  i = plsc.bitcast(d, jnp.int32)
  i = 0x5F3759DF - (i >> 1)
  r = plsc.bitcast(i, jnp.float32)
  for _ in range(4):
    r = r * (1.5 - 0.5 * d * r * r)
  return jnp.where(d > 0.0, r, 1.0)


def _mm_body(x_ref, w_ref, b_ref, o_ref):
  o_ref[0] = (
      jnp.dot(x_ref[...], w_ref[0], preferred_element_type=jnp.float32)
      + b_ref[0]
  )


def _sc_body(h0_hbm, e_hbm, out_hbm,
             hs_sp, agg_sp, degs_sp, degd_sp,
             ibuf, gbuf0, gbuf1, gbuf2, gbuf3, nbuf,
             ones_v, ns_v, nd_v,
             isem, gsem, ssem):
  cid = lax.axis_index("c")
  sid = lax.axis_index("s")
  n0 = sid * ROWS_PER_TILE
  gbufs = (gbuf0, gbuf1, gbuf2, gbuf3)

  zeros16 = jnp.zeros((16,), jnp.float32)
  ones16 = jnp.ones((16,), jnp.float32)

  def _fill_gbuf2(r, c):
    for k in range(DH // 16):
      gbuf2[r, pl.ds(16 * k, 16)] = zeros16
    return c
  lax.fori_loop(0, NQ, _fill_gbuf2, 0)

  def _fill_norms0(q, c):
    ns_v[pl.ds(16 * q, 16)] = zeros16
    nd_v[pl.ds(16 * q, 16)] = zeros16
    return c
  lax.fori_loop(0, ROWS_PER_TILE // 16, _fill_norms0, 0)

  for k in range(EC // 16):
    ones_v[pl.ds(16 * k, 16)] = ones16

  rows640 = pl.ds(n0, ROWS_PER_TILE)
  nsl640 = pl.ds(0, ROWS_PER_TILE)

  for q5 in range(NQCHUNKS):
    pltpu.async_copy(gbuf2, agg_sp.at[pl.ds(n0 + NQ * q5, NQ)], gsem.at[0])
  pltpu.async_copy(ns_v.at[nsl640], degs_sp.at[rows640], gsem.at[1])
  pltpu.async_copy(nd_v.at[nsl640], degd_sp.at[rows640], gsem.at[1])

  for q5 in range(NQCHUNKS):
    pltpu.make_async_copy(gbuf2, agg_sp.at[pl.ds(n0, NQ)],
                          gsem.at[0]).wait()
  pltpu.make_async_copy(ns_v.at[nsl640], degs_sp.at[rows640],
                        gsem.at[1]).wait()
  pltpu.make_async_copy(nd_v.at[nsl640], degd_sp.at[rows640],
                        gsem.at[1]).wait()

  plsc.subcore_barrier()

  def _idx_start(jj, u):
    pltpu.async_copy(e_hbm.at[0, sid, jj], ibuf.at[u, 0], isem.at[u])
    pltpu.async_copy(e_hbm.at[1, sid, jj], ibuf.at[u, 1], isem.at[u])

  def _idx_wait(jj, u):
    pltpu.make_async_copy(
        e_hbm.at[0, sid, jj], ibuf.at[u, 0], isem.at[u]).wait()
    pltpu.make_async_copy(
        e_hbm.at[1, sid, jj], ibuf.at[u, 1], isem.at[u]).wait()

  for s in range(PDIST):
    _idx_start(s, s)

  def _deg_body(g, c):
    for u in range(UNROLL):
      j = UNROLL * g + u
      b = u % 4
      if u < 4:
        @pl.when(g > 0)
        def _():
          pltpu.make_async_copy(
              ones_v, degs_sp.at[ibuf.at[(u - 4) % UNROLL, 0]],
              gsem.at[b]).wait()
          pltpu.make_async_copy(
              ones_v, degd_sp.at[ibuf.at[(u - 4) % UNROLL, 1]],
              ssem.at[b]).wait()
      else:
        pltpu.make_async_copy(
            ones_v, degs_sp.at[ibuf.at[u - 4, 0]], gsem.at[b]).wait()
        pltpu.make_async_copy(
            ones_v, degd_sp.at[ibuf.at[u - 4, 1]], ssem.at[b]).wait()
      if u < UNROLL - PDIST:
        _idx_start(j + PDIST, u + PDIST)
      else:
        @pl.when(g < GROUPS - 1)
        def _():
          _idx_start(j + PDIST, u + PDIST - UNROLL)
      _idx_wait(j, u)
      pltpu.async_copy(ones_v, degs_sp.at[ibuf.at[u, 0]], gsem.at[b],
                       add=True)
      pltpu.async_copy(ones_v, degd_sp.at[ibuf.at[u, 1]], ssem.at[b],
                       add=True)
    return c
  lax.fori_loop(0, GROUPS, _deg_body, 0)
  for b in range(4):
    u = UNROLL - 4 + b
    pltpu.make_async_copy(
        ones_v, degs_sp.at[ibuf.at[u, 0]], gsem.at[b]).wait()
    pltpu.make_async_copy(
        ones_v, degd_sp.at[ibuf.at[u, 1]], ssem.at[b]).wait()

  plsc.subcore_barrier()

  pltpu.sync_copy(degs_sp.at[rows640], ns_v.at[pl.ds(0, ROWS_PER_TILE)])
  pltpu.sync_copy(degd_sp.at[rows640], nd_v.at[pl.ds(0, ROWS_PER_TILE)])

  def _norm_body(q, c):
    sl = pl.ds(16 * q, 16)
    ns_v[sl] = _rsqrt_pos(ns_v[sl])
    nd_v[sl] = _rsqrt_pos(nd_v[sl])
    return c
  lax.fori_loop(0, ROWS_PER_TILE // 16, _norm_body, 0)

  h0bufs = (gbuf0, gbuf1)

  def _h0_start(q, p):
    pltpu.async_copy(h0_hbm.at[cid, pl.ds(n0 + NQ * q, NQ)], h0bufs[p],
                     isem.at[2 + p])

  def _h0_wait(q, p):
    pltpu.make_async_copy(h0_hbm.at[cid, pl.ds(n0, NQ)], h0bufs[p],
                          isem.at[2 + p]).wait()

  def _hs_wait():
    pltpu.make_async_copy(nbuf, hs_sp.at[pl.ds(n0, NQ)], isem.at[4]).wait()

  _h0_start(0, 0)
  for q in range(NQCHUNKS):
    p = q % 2
    if q + 1 < NQCHUNKS:
      _h0_start(q + 1, 1 - p)
    _h0_wait(q, p)

    def _hs0_body(g8, c, q=q, p=p):
      nsv = ns_v[pl.ds(NQ * q + 8 * g8, 16)]
      for i in range(8):
        r = 8 * g8 + i
        ns_s = nsv[i]
        for k in range(DH // 16):
          sl = pl.ds(16 * k, 16)
          nbuf[r, sl] = h0bufs[p][r, sl] * ns_s
      return c
    lax.fori_loop(0, NQ // 8, _hs0_body, 0)
    pltpu.sync_copy(nbuf, hs_sp.at[pl.ds(n0 + NQ * q, NQ)])

  plsc.subcore_barrier()

  def _gather_start(jj, u, b):
    pltpu.async_copy(hs_sp.at[ibuf.at[u, 0]], gbufs[b], gsem.at[b])

  def _gather_wait(jj, u, b):
    pltpu.make_async_copy(
        hs_sp.at[ibuf.at[u, 0]], gbufs[b], gsem.at[b]).wait()

  def _scatter_start(jj, u, b):
    pltpu.async_copy(gbufs[b], agg_sp.at[ibuf.at[u, 1]], ssem.at[b],
                     add=True)

  def _scatter_wait(jj, u, b):
    pltpu.make_async_copy(
        gbufs[b], agg_sp.at[ibuf.at[u, 1]], ssem.at[b]).wait()

  def _one_round(last):
    t = K - 1 if last else 0
    for s in range(PDIST):
      _idx_start(s, s)

    def _edge_body(g, c):
      for u in range(UNROLL):
        j = UNROLL * g + u
        b = u % 4
        if u < 4:
          @pl.when(g > 0)
          def _():
            _scatter_wait(j - 4, (u - 4) % UNROLL, b)
        else:
          _scatter_wait(j - 4, u - 4, b)
        if u < UNROLL - PDIST:
          _idx_start(j + PDIST, u + PDIST)
        else:
          @pl.when(g < GROUPS - 1)
          def _():
            _idx_start(j + PDIST, u + PDIST - UNROLL)
        _idx_wait(j, u)
        _gather_start(j, u, b)
        up = (u - 1) % UNROLL
        bp = up % 4
        if u == 0:
          @pl.when(g > 0)
          def _():
            _gather_wait(j - 1, up, bp)
            _scatter_start(j - 1, up, bp)
        else:
          _gather_wait(j - 1, up, bp)
          _scatter_start(j - 1, up, bp)
      return c

    lax.fori_loop(0, GROUPS, _edge_body, 0)
    _gather_wait(ECHUNKS - 1, UNROLL - 1, 3)
    _scatter_start(ECHUNKS - 1, UNROLL - 1, 3)
    for b in range(4):
      _scatter_wait(ECHUNKS - 4 + b, UNROLL - 4 + b, b)

    plsc.subcore_barrier()

    nbufs = (nbuf, gbuf3)

    if t < K - 1:
      def _rezero(r, c):
        for k in range(DH // 16):
          gbuf2[r, pl.ds(16 * k, 16)] = zeros16
        return c
      lax.fori_loop(0, NQ, _rezero, 0)

    def _agg_start(q, p):
      pltpu.async_copy(agg_sp.at[pl.ds(n0 + NQ * q, NQ)], nbufs[p],
                       isem.at[p])

    def _agg_wait(q, p):
      pltpu.make_async_copy(agg_sp.at[pl.ds(n0, NQ)], nbufs[p],
                            isem.at[p]).wait()

    _agg_start(0, 0)
    _h0_start(0, 0)
    for q in range(NQCHUNKS):
      p = q % 2
      rows = pl.ds(n0 + NQ * q, NQ)
      if q >= 2:
        pass
      if q + 1 < NQCHUNKS:
        if q >= 1:
          if t < K - 1:
            pltpu.make_async_copy(
                nbufs[1 - p], hs_sp.at[pl.ds(n0, NQ)],
                isem.at[4 + (1 - p)]).wait()
        _agg_start(q + 1, 1 - p)
        _h0_start(q + 1, 1 - p)
      _agg_wait(q, p)
      _h0_wait(q, p)

      def _node_body(g8, c, q=q, t=t, p=p):
        ndv = nd_v[pl.ds(NQ * q + 8 * g8, 16)]
        nsv = ns_v[pl.ds(NQ * q + 8 * g8, 16)]
        for i in range(8):
          r = 8 * g8 + i
          nd_s = C_AGG * ndv[i]
          ns_s = nsv[i]
          for k in range(DH // 16):
            sl = pl.ds(16 * k, 16)
            hn = nd_s * nbufs[p][r, sl] + C_H0 * h0bufs[p][r, sl]
            if t < K - 1:
              nbufs[p][r, sl] = hn * ns_s
            else:
              nbufs[p][r, sl] = hn
        return c
      lax.fori_loop(0, NQ // 8, _node_body, 0)

      if t < K - 1:
        pltpu.async_copy(gbuf2, agg_sp.at[rows], isem.at[6])
        pltpu.async_copy(nbufs[p], hs_sp.at[rows], isem.at[4 + p])
      else:
        cols = pl.ds(cid * DH, DH)
        nrows15 = min(max(N - (NS - 1) * ROWS_PER_TILE - NQ * q, 0), NQ)
        @pl.when(sid < NS - 1)
        def _(q=q, cols=cols, p=p, rows=rows):
          pltpu.sync_copy(nbufs[p], out_hbm.at[rows, cols])
        if nrows15 > 0:
          @pl.when(sid == NS - 1)
          def _(q=q, cols=cols, nrows15=nrows15, p=p):
            pltpu.sync_copy(
                nbufs[p].at[pl.ds(0, nrows15)],
                out_hbm.at[pl.ds((NS - 1) * ROWS_PER_TILE + NQ * q,
                                 nrows15), cols])

    if t < K - 1:
      for p in (NQCHUNKS % 2, 1 - NQCHUNKS % 2):
        pltpu.make_async_copy(
            nbufs[p], hs_sp.at[pl.ds(n0, NQ)], isem.at[4 + p]).wait()
      for _z in range(NQCHUNKS):
        pltpu.make_async_copy(
            gbuf2, agg_sp.at[pl.ds(n0, NQ)], isem.at[6]).wait()
      plsc.subcore_barrier()

  def _round_body(t, c):
    _one_round(False)
    return c
  lax.fori_loop(0, K - 1, _round_body, 0)
  _one_round(True)


@jax.jit
def kernel(x, edge_index, W, b):
  w_split = W.reshape(D, NC, DH).transpose(1, 0, 2)
  b_split = b.reshape(1, NC, DH).transpose(1, 0, 2)
  h0_split = pl.pallas_call(
      _mm_body,
      grid=(NPAD // MM_BLOCK, NC),
      in_specs=[
          pl.BlockSpec((MM_BLOCK, D), lambda i, c: (i, 0)),
          pl.BlockSpec((1, D, DH), lambda i, c: (c, 0, 0)),
          pl.BlockSpec((1, 1, DH), lambda i, c: (c, 0, 0)),
      ],
      out_specs=pl.BlockSpec((1, MM_BLOCK, DH), lambda i, c: (c, i, 0)),
      out_shape=jax.ShapeDtypeStruct((NC, NPAD, DH), jnp.float32),
  )(x, w_split, b_split)

  e4 = jnp.pad(edge_index, ((0, 0), (0, E_PAD - E)),
               constant_values=SENT).reshape(2, NS, ECHUNKS, EC)

  mesh = plsc.VectorSubcoreMesh(
      core_axis_name="c", subcore_axis_name="s",
      num_cores=NC, num_subcores=NS)

  sc = pl.kernel(
      _sc_body,
      out_type=jax.ShapeDtypeStruct((N, D), jnp.float32),
      mesh=mesh,
      compiler_params=pltpu.CompilerParams(
          needs_layout_passes=False, use_tc_tiling_on_sc=False),
      scratch_types=[
          pltpu.VMEM_SHARED((NPAD, DH), jnp.float32),
          pltpu.VMEM_SHARED((NPAD, DH), jnp.float32),
          pltpu.VMEM_SHARED((NPAD,), jnp.float32),
          pltpu.VMEM_SHARED((NPAD,), jnp.float32),
          pltpu.VMEM((UNROLL, 2, EC), jnp.int32),
          pltpu.VMEM((EC, DH), jnp.float32),
          pltpu.VMEM((EC, DH), jnp.float32),
          pltpu.VMEM((EC, DH), jnp.float32),
          pltpu.VMEM((EC, DH), jnp.float32),
          pltpu.VMEM((NQ, DH), jnp.float32),
          pltpu.VMEM((EC,), jnp.float32),
          pltpu.VMEM((ROWS_PER_TILE + 16,), jnp.float32),
          pltpu.VMEM((ROWS_PER_TILE + 16,), jnp.float32),
          pltpu.SemaphoreType.DMA((UNROLL,)),
          pltpu.SemaphoreType.DMA((4,)),
          pltpu.SemaphoreType.DMA((4,)),
      ],
  )

  return sc(h0_split, e4)
